# Initial kernel scaffold; baseline (speedup 1.0000x reference)
import functools
import jax
import jax.numpy as jnp
from jax import lax
from jax.experimental import pallas as pl
from jax.experimental.pallas import tpu as pltpu
from jax.experimental.pallas import tpu_sc as plsc

N = 100000
DIM = 32
HALF = 50000
CH = 128
ROWS = 12512
E_PAD = ROWS * CH
SENT = 100000
HPAD = 100352
HSL = HPAD // 16
APAD = 51200
TRASH = HALF

A_ROWS = ROWS // 32
C_ROWS = ROWS // 16
CK = 34
CG = C_ROWS // CK

RB = 6272

_mesh = plsc.VectorSubcoreMesh(core_axis_name="c", subcore_axis_name="s")


@functools.partial(
    pl.kernel,
    out_type=(
        jax.ShapeDtypeStruct((2, HPAD, 1), jnp.float32),
        jax.ShapeDtypeStruct((2, HPAD, 1), jnp.float32),
    ),
    mesh=_mesh,
    scratch_types=[
        pltpu.VMEM((A_ROWS, CH), jnp.int32),
        pltpu.VMEM((A_ROWS, CH), jnp.int32),
        pltpu.VMEM((CH, 1), jnp.float32),
        pltpu.VMEM((HSL, 1), jnp.float32),
        pltpu.VMEM_SHARED((HPAD, 1), jnp.float32),
        pltpu.VMEM_SHARED((HPAD, 1), jnp.float32),
    ],
)
def _deg_kernel(src_hbm, dst_hbm, ones_hbm, zcol_hbm, outd_hbm, ind_hbm,
                srcb, dstb, onesb, colb, outh, inh):
    c = lax.axis_index("c")
    s = lax.axis_index("s")
    w = s * 2 + c

    pltpu.sync_copy(zcol_hbm, colb)
    pltpu.sync_copy(colb, outh.at[pl.ds(s * HSL, HSL)])
    pltpu.sync_copy(colb, inh.at[pl.ds(s * HSL, HSL)])
    pltpu.sync_copy(ones_hbm, onesb)
    pltpu.sync_copy(src_hbm.at[pl.ds(w * A_ROWS, A_ROWS)], srcb)
    pltpu.sync_copy(dst_hbm.at[pl.ds(w * A_ROWS, A_ROWS)], dstb)
    plsc.subcore_barrier()

    def body(j, carry):
        pltpu.sync_copy(onesb, outh.at[srcb.at[j]], add=True)
        pltpu.sync_copy(onesb, inh.at[dstb.at[j]], add=True)
        return carry

    lax.fori_loop(0, A_ROWS, body, 0)
    plsc.subcore_barrier()

    pltpu.sync_copy(outh.at[pl.ds(s * HSL, HSL)], colb)
    pltpu.sync_copy(colb, outd_hbm.at[c, pl.ds(s * HSL, HSL)])
    pltpu.sync_copy(inh.at[pl.ds(s * HSL, HSL)], colb)
    pltpu.sync_copy(colb, ind_hbm.at[c, pl.ds(s * HSL, HSL)])


@functools.partial(
    pl.kernel,
    out_type=jax.ShapeDtypeStruct((N, DIM), jnp.float32),
    mesh=_mesh,
    scratch_types=[
        pltpu.VMEM((CK, CH), jnp.int32),
        pltpu.VMEM((CK, CH), jnp.int32),
        pltpu.VMEM((CK, CH), jnp.int32),
        pltpu.VMEM((CH, DIM), jnp.float32),
        pltpu.VMEM((640, DIM), jnp.float32),
        pltpu.VMEM_SHARED((APAD, DIM), jnp.float32),
    ],
)
def _agg_kernel(h_hbm, src_hbm, dst_hbm, zrow_hbm, out_hbm,
                srcb, dstb, locb, rowsb, stage, acc):
    c = lax.axis_index("c")
    s = lax.axis_index("s")
    base = c * HALF

    pltpu.sync_copy(zrow_hbm, stage)
    for k in range(5):
        pltpu.sync_copy(stage, acc.at[pl.ds(s * 3200 + k * 640, 640)])
    plsc.subcore_barrier()

    def group(g, carry):
        r0 = s * C_ROWS + g * CK
        pltpu.sync_copy(src_hbm.at[pl.ds(r0, CK)], srcb)
        pltpu.sync_copy(dst_hbm.at[pl.ds(r0, CK)], dstb)

        def crow(j, cc):
            def cvec(i, ci):
                d = dstb[j, pl.ds(i * 16, 16)]
                loc = d - base
                ok = (loc >= 0) & (loc < HALF)
                locb[j, pl.ds(i * 16, 16)] = jnp.where(ok, loc, TRASH)
                return ci
            return lax.fori_loop(0, CH // 16, cvec, cc)

        lax.fori_loop(0, CK, crow, 0)

        def arow(j, cc):
            pltpu.sync_copy(h_hbm.at[srcb.at[j]], rowsb)
            pltpu.sync_copy(rowsb, acc.at[locb.at[j]], add=True)
            return cc

        lax.fori_loop(0, CK, arow, 0)
        return carry

    lax.fori_loop(0, CG, group, 0)
    plsc.subcore_barrier()

    for k in range(5):
        r = s * 3125 + k * 625
        pltpu.sync_copy(acc.at[pl.ds(r, 625)], stage.at[pl.ds(0, 625)])
        pltpu.sync_copy(stage.at[pl.ds(0, 625)],
                        out_hbm.at[pl.ds(base + r, 625)])


def _norm_body(x_ref, od_ref, id_ref, h_ref, nd_ref):
    od = od_ref[0, :, :] + od_ref[1, :, :]
    idg = id_ref[0, :, :] + id_ref[1, :, :]
    ns = jnp.where(od > 0, lax.rsqrt(jnp.maximum(od, 1.0)), 0.0)
    nd = jnp.where(idg > 0, lax.rsqrt(jnp.maximum(idg, 1.0)), 0.0)
    h_ref[...] = x_ref[...] * ns
    nd_ref[...] = nd


_norm_call = pl.pallas_call(
    _norm_body,
    grid=(16,),
    in_specs=[
        pl.BlockSpec((RB, DIM), lambda i: (i, 0)),
        pl.BlockSpec((2, RB, 1), lambda i: (0, i, 0)),
        pl.BlockSpec((2, RB, 1), lambda i: (0, i, 0)),
    ],
    out_specs=(
        pl.BlockSpec((RB, DIM), lambda i: (i, 0)),
        pl.BlockSpec((RB, 1), lambda i: (i, 0)),
    ),
    out_shape=(
        jax.ShapeDtypeStruct((HPAD, DIM), jnp.float32),
        jax.ShapeDtypeStruct((HPAD, 1), jnp.float32),
    ),
)


def _proj_body(agg_ref, nd_ref, w_ref, b_ref, o_ref):
    a = agg_ref[...] * nd_ref[...]
    o_ref[...] = jnp.dot(a, w_ref[...],
                         preferred_element_type=jnp.float32) + b_ref[...]


_proj_call = pl.pallas_call(
    _proj_body,
    grid=(16,),
    in_specs=[
        pl.BlockSpec((RB, DIM), lambda i: (i, 0)),
        pl.BlockSpec((RB, 1), lambda i: (i, 0)),
        pl.BlockSpec((DIM, DIM), lambda i: (0, 0)),
        pl.BlockSpec((1, DIM), lambda i: (0, 0)),
    ],
    out_specs=pl.BlockSpec((RB, DIM), lambda i: (i, 0)),
    out_shape=jax.ShapeDtypeStruct((N, DIM), jnp.float32),
)


def kernel(x, edge_index, W, b):
    src = edge_index[0].astype(jnp.int32)
    dst = edge_index[1].astype(jnp.int32)
    pad = jnp.full((E_PAD - src.shape[0],), SENT, jnp.int32)
    src2 = jnp.concatenate([src, pad]).reshape(ROWS, CH)
    dst2 = jnp.concatenate([dst, pad]).reshape(ROWS, CH)
    ones_col = jnp.ones((CH, 1), jnp.float32)
    zcol = jnp.zeros((HSL, 1), jnp.float32)
    zrow = jnp.zeros((640, DIM), jnp.float32)

    outd, ind = _deg_kernel(src2, dst2, ones_col, zcol)
    h, nd = _norm_call(x, outd, ind)
    agg = _agg_kernel(h, src2, dst2, zrow)
    out = _proj_call(agg, nd, W, b.reshape(1, DIM))
    return out


# speedup vs baseline: 8.0966x; 8.0966x over previous
import functools
import jax
import jax.numpy as jnp
from jax import lax
from jax.experimental import pallas as pl
from jax.experimental.pallas import tpu as pltpu
from jax.experimental.pallas import tpu_sc as plsc

N = 100000
DIM = 32
HALF = 50000
CH = 128
ROWS = 12544
E_PAD = ROWS * CH
SENT = 100000
HPAD = 100352
HSL = HPAD // 16
HW = 16
HSW = 392
AK = 56
AG = 7
APAD = 50048
TRASH = HALF
NWCH = APAD // CH

A_ROWS = ROWS // 32
C_ROWS = ROWS // 16
CK = 16
CG = C_ROWS // CK

RB = 6272

_mesh = plsc.VectorSubcoreMesh(core_axis_name="c", subcore_axis_name="s")


@functools.partial(
    pl.kernel,
    out_type=jax.ShapeDtypeStruct((2, HPAD, HW), jnp.float32),
    mesh=_mesh,
    compiler_params=pltpu.CompilerParams(use_tc_tiling_on_sc=False),
    scratch_types=[
        pltpu.VMEM((AK, CH), jnp.int32),
        pltpu.VMEM((AK, CH), jnp.int32),
        pltpu.VMEM((CH, HW), jnp.float32),
        pltpu.VMEM((CH, HW), jnp.float32),
        pltpu.VMEM((HSW, HW), jnp.float32),
        pltpu.VMEM_SHARED((HPAD, HW), jnp.float32),
    ],
)
def _deg_kernel(src_hbm, dst_hbm, onl_hbm, onr_hbm, zcol_hbm, deg_hbm,
                srcb, dstb, onlb, onrb, slabb, hist):
    c = lax.axis_index("c")
    s = lax.axis_index("s")
    w = s * 2 + c

    pltpu.sync_copy(zcol_hbm, slabb)
    for k in range(16):
        pltpu.sync_copy(slabb, hist.at[pl.ds(s * HSL + k * HSW, HSW)])
    pltpu.sync_copy(onl_hbm, onlb)
    pltpu.sync_copy(onr_hbm, onrb)
    plsc.subcore_barrier()

    def group(g, carry):
        r0 = w * A_ROWS + g * AK
        pltpu.sync_copy(src_hbm.at[pl.ds(r0, AK)], srcb)
        pltpu.sync_copy(dst_hbm.at[pl.ds(r0, AK)], dstb)

        def body(j, cc):
            pltpu.sync_copy(onlb, hist.at[srcb.at[j]], add=True)
            pltpu.sync_copy(onrb, hist.at[dstb.at[j]], add=True)
            return cc

        lax.fori_loop(0, AK, body, 0)
        return carry

    lax.fori_loop(0, AG, group, 0)
    plsc.subcore_barrier()

    for k in range(16):
        r = s * HSL + k * HSW
        pltpu.sync_copy(hist.at[pl.ds(r, HSW)], slabb)
        pltpu.sync_copy(slabb, deg_hbm.at[c, pl.ds(r, HSW)])


@functools.partial(
    pl.kernel,
    out_type=jax.ShapeDtypeStruct((N, DIM), jnp.float32),
    mesh=_mesh,
    compiler_params=pltpu.CompilerParams(use_tc_tiling_on_sc=False),
    scratch_types=[
        pltpu.VMEM((CK, CH), jnp.int32),
        pltpu.VMEM((CK, CH), jnp.int32),
        pltpu.VMEM((CK, CH), jnp.int32),
        pltpu.VMEM((CH, DIM), jnp.float32),
        pltpu.VMEM_SHARED((APAD, DIM), jnp.float32),
    ],
)
def _agg_kernel(h_hbm, src_hbm, dst_hbm, zrow_hbm, out_hbm,
                srcb, dstb, locb, rowsb, acc):
    c = lax.axis_index("c")
    s = lax.axis_index("s")
    base = c * HALF

    pltpu.sync_copy(zrow_hbm, rowsb)
    for k in range(25):
        ch = s + 16 * k

        @pl.when(ch < NWCH)
        def _():
            pltpu.sync_copy(rowsb, acc.at[pl.ds(ch * CH, CH)])

    plsc.subcore_barrier()

    def group(g, carry):
        r0 = s * C_ROWS + g * CK
        pltpu.sync_copy(src_hbm.at[pl.ds(r0, CK)], srcb)
        pltpu.sync_copy(dst_hbm.at[pl.ds(r0, CK)], dstb)

        def crow(j, cc):
            def cvec(i, ci):
                d = dstb[j, pl.ds(i * 16, 16)]
                loc = d - base
                ok = (loc >= 0) & (loc < HALF)
                locb[j, pl.ds(i * 16, 16)] = jnp.where(ok, loc, TRASH)
                return ci
            return lax.fori_loop(0, CH // 16, cvec, cc)

        lax.fori_loop(0, CK, crow, 0)

        def arow(j, cc):
            pltpu.sync_copy(h_hbm.at[srcb.at[j]], rowsb)
            pltpu.sync_copy(rowsb, acc.at[locb.at[j]], add=True)
            return cc

        lax.fori_loop(0, CK, arow, 0)
        return carry

    lax.fori_loop(0, CG, group, 0)
    plsc.subcore_barrier()

    for k in range(25):
        ch = s + 16 * k

        @pl.when(ch < NWCH - 1)
        def _():
            pltpu.sync_copy(acc.at[pl.ds(ch * CH, CH)], rowsb)
            pltpu.sync_copy(rowsb, out_hbm.at[pl.ds(base + ch * CH, CH)])

        @pl.when(ch == NWCH - 1)
        def _():
            pltpu.sync_copy(acc.at[pl.ds(ch * CH, 80)],
                            rowsb.at[pl.ds(0, 80)])
            pltpu.sync_copy(rowsb.at[pl.ds(0, 80)],
                            out_hbm.at[pl.ds(base + ch * CH, 80)])


def _norm_body(x_ref, deg_ref, h_ref, nd_ref):
    od = deg_ref[0, :, 0:1] + deg_ref[1, :, 0:1]
    idg = deg_ref[0, :, 8:9] + deg_ref[1, :, 8:9]
    ns = jnp.where(od > 0, lax.rsqrt(jnp.maximum(od, 1.0)), 0.0)
    nd = jnp.where(idg > 0, lax.rsqrt(jnp.maximum(idg, 1.0)), 0.0)
    h_ref[...] = x_ref[...] * ns
    nd_ref[...] = nd


_norm_call = pl.pallas_call(
    _norm_body,
    grid=(16,),
    in_specs=[
        pl.BlockSpec((RB, DIM), lambda i: (i, 0)),
        pl.BlockSpec((2, RB, HW), lambda i: (0, i, 0)),
    ],
    out_specs=(
        pl.BlockSpec((RB, DIM), lambda i: (i, 0)),
        pl.BlockSpec((RB, 1), lambda i: (i, 0)),
    ),
    out_shape=(
        jax.ShapeDtypeStruct((HPAD, DIM), jnp.float32),
        jax.ShapeDtypeStruct((HPAD, 1), jnp.float32),
    ),
)


def _proj_body(agg_ref, nd_ref, w_ref, b_ref, o_ref):
    a = agg_ref[...] * nd_ref[...]
    o_ref[...] = jnp.dot(a, w_ref[...],
                         preferred_element_type=jnp.float32) + b_ref[...]


_proj_call = pl.pallas_call(
    _proj_body,
    grid=(16,),
    in_specs=[
        pl.BlockSpec((RB, DIM), lambda i: (i, 0)),
        pl.BlockSpec((RB, 1), lambda i: (i, 0)),
        pl.BlockSpec((DIM, DIM), lambda i: (0, 0)),
        pl.BlockSpec((1, DIM), lambda i: (0, 0)),
    ],
    out_specs=pl.BlockSpec((RB, DIM), lambda i: (i, 0)),
    out_shape=jax.ShapeDtypeStruct((N, DIM), jnp.float32),
)


def kernel(x, edge_index, W, b):
    src = edge_index[0].astype(jnp.int32)
    dst = edge_index[1].astype(jnp.int32)
    pad = jnp.full((E_PAD - src.shape[0],), SENT, jnp.int32)
    src2 = jnp.concatenate([src, pad]).reshape(ROWS, CH)
    dst2 = jnp.concatenate([dst, pad]).reshape(ROWS, CH)
    col = jnp.arange(HW)
    ones_l = jnp.where(col < 8, 1.0, 0.0).astype(jnp.float32)
    ones_l = jnp.broadcast_to(ones_l, (CH, HW))
    ones_r = jnp.where(col >= 8, 1.0, 0.0).astype(jnp.float32)
    ones_r = jnp.broadcast_to(ones_r, (CH, HW))
    zcol = jnp.zeros((HSW, HW), jnp.float32)
    zrow = jnp.zeros((CH, DIM), jnp.float32)

    deg = _deg_kernel(src2, dst2, ones_l, ones_r, zcol)
    h, nd = _norm_call(x, deg)
    agg = _agg_kernel(h, src2, dst2, zrow)
    out = _proj_call(agg, nd, W, b.reshape(1, DIM))
    return out
